# Initial kernel scaffold; baseline (speedup 1.0000x reference)
import jax
import jax.numpy as jnp
from jax.experimental import pallas as pl


def kernel(x, edge_index, W1, b1, W2, b2, W3, b3, Wmu, bmu, Wls, bls):
    raise NotImplementedError("write your pallas kernel here")


# speedup vs baseline: 12.7433x; 12.7433x over previous
import functools

import jax
import jax.numpy as jnp
from jax import lax
from jax.experimental import pallas as pl
from jax.experimental.pallas import tpu as pltpu
from jax.experimental.pallas import tpu_sc as plsc

N = 10000
NR = 10240
F = 64
E = 320000
CH = 128
NW = 32
NCH = 80
EPT = NCH * CH
E_PAD = NW * EPT
RPS = NR // 16

_MESH = plsc.VectorSubcoreMesh(
    core_axis_name="c", subcore_axis_name="s", num_cores=2, num_subcores=16
)
_SC_PARAMS = pltpu.CompilerParams(use_tc_tiling_on_sc=False)


@functools.partial(
    pl.kernel,
    out_type=jax.ShapeDtypeStruct((2, NR, 16), jnp.float32),
    mesh=_MESH,
    scratch_types=[
        pltpu.VMEM((NCH, CH), jnp.int32),
        pltpu.VMEM((CH, 16), jnp.float32),
        pltpu.VMEM_SHARED((NR, 16), jnp.float32),
    ],
    compiler_params=_SC_PARAMS,
)
def _deg_counts(dsts_hbm, zeros_hbm, ones_hbm, out_hbm, dst_v, ones_v, acc):
    c = lax.axis_index("c")
    s = lax.axis_index("s")
    w = c * 16 + s
    pltpu.sync_copy(zeros_hbm.at[pl.ds(s * RPS, RPS)], acc.at[pl.ds(s * RPS, RPS)])
    pltpu.sync_copy(dsts_hbm.at[w], dst_v)
    pltpu.sync_copy(ones_hbm, ones_v)
    plsc.subcore_barrier()

    @pl.loop(0, NCH)
    def _(j):
        pltpu.sync_copy(ones_v, acc.at[dst_v.at[j]], add=True)

    plsc.subcore_barrier()
    pltpu.sync_copy(acc.at[pl.ds(s * RPS, RPS)], out_hbm.at[c, pl.ds(s * RPS, RPS)])


@functools.partial(
    pl.kernel,
    out_type=jax.ShapeDtypeStruct((2, NR, F), jnp.float32),
    mesh=_MESH,
    scratch_types=[
        pltpu.VMEM((NCH, CH), jnp.int32),
        pltpu.VMEM((NCH, CH), jnp.int32),
        pltpu.VMEM((CH, F), jnp.float32),
        pltpu.VMEM((CH, F), jnp.float32),
        pltpu.VMEM_SHARED((NR, F), jnp.float32),
        pltpu.SemaphoreType.DMA,
        pltpu.SemaphoreType.DMA,
    ],
    compiler_params=_SC_PARAMS,
)
def _agg_pass(z_hbm, srcs_hbm, dsts_hbm, out_hbm,
              src_v, dst_v, rows0, rows1, acc, sem0, sem1):
    c = lax.axis_index("c")
    s = lax.axis_index("s")
    w = c * 16 + s
    pltpu.sync_copy(z_hbm.at[pl.ds(s * RPS, RPS)], acc.at[pl.ds(s * RPS, RPS)])
    pltpu.sync_copy(srcs_hbm.at[w], src_v)
    pltpu.sync_copy(dsts_hbm.at[w], dst_v)
    plsc.subcore_barrier()

    @pl.loop(0, NCH, step=2)
    def _(j):
        cpa = pltpu.async_copy(z_hbm.at[src_v.at[j]], rows0, sem0)
        cpb = pltpu.async_copy(z_hbm.at[src_v.at[j + 1]], rows1, sem1)
        cpa.wait()
        pltpu.sync_copy(rows0, acc.at[dst_v.at[j]], add=True)
        cpb.wait()
        pltpu.sync_copy(rows1, acc.at[dst_v.at[j + 1]], add=True)

    plsc.subcore_barrier()
    pltpu.sync_copy(acc.at[pl.ds(s * RPS, RPS)], out_hbm.at[c, pl.ds(s * RPS, RPS)])


BLK = 2048
GRID = NR // BLK

def _dinv(cnt):
    return lax.rsqrt(1.0 + cnt[0, :, :1] + cnt[1, :, :1])


def _body_a(cnt_ref, x_ref, w1_ref, z1_ref):
    d = _dinv(cnt_ref[...])
    t = jnp.dot(x_ref[...], w1_ref[...], preferred_element_type=jnp.float32)
    z1_ref[...] = d * t


def _body_b(cnt_ref, u_ref, z_ref, b_ref, out_ref):
    d = _dinv(cnt_ref[...])
    u = u_ref[...]
    h = jnp.maximum(d * (u[0] + u[1] - z_ref[...]) + b_ref[...], 0.0)
    out_ref[...] = d * h


def _body_c(cnt_ref, u_ref, z_ref, w2_ref, b2_ref, w3_ref, out_ref):
    d = _dinv(cnt_ref[...])
    u = u_ref[...]
    a2 = d * (u[0] + u[1] - z_ref[...])
    h2 = jnp.maximum(
        jnp.dot(a2, w2_ref[...], preferred_element_type=jnp.float32) + b2_ref[...], 0.0)
    t3 = jnp.dot(h2, w3_ref[...], preferred_element_type=jnp.float32)
    out_ref[...] = d * t3


def _body_e(cnt_ref, u_ref, z_ref, wmu_ref, bmu_ref, wls_ref, bls_ref,
            mu_ref, ls_ref):
    d = _dinv(cnt_ref[...])
    u = u_ref[...]
    a4 = d * (u[0] + u[1] - z_ref[...])
    mu_ref[...] = jnp.dot(a4, wmu_ref[...], preferred_element_type=jnp.float32) + bmu_ref[...]
    ls_ref[...] = jnp.dot(a4, wls_ref[...], preferred_element_type=jnp.float32) + bls_ref[...]


def _row_spec(width):
    return pl.BlockSpec((BLK, width), lambda i: (i, 0))


def _cnt_spec():
    return pl.BlockSpec((2, BLK, 16), lambda i: (0, i, 0))


def _u_spec():
    return pl.BlockSpec((2, BLK, F), lambda i: (0, i, 0))


def _full_spec(shape):
    nd = len(shape)
    return pl.BlockSpec(shape, lambda i: (0,) * nd)


def _tc_a(cnt, x, w1):
    return pl.pallas_call(
        _body_a,
        grid=(GRID,),
        in_specs=[_cnt_spec(), _row_spec(128), _full_spec((128, F))],
        out_specs=_row_spec(F),
        out_shape=jax.ShapeDtypeStruct((NR, F), jnp.float32),
    )(cnt, x, w1)


def _tc_b(cnt, u, z, b):
    return pl.pallas_call(
        _body_b,
        grid=(GRID,),
        in_specs=[_cnt_spec(), _u_spec(), _row_spec(F), _full_spec((1, F))],
        out_specs=_row_spec(F),
        out_shape=jax.ShapeDtypeStruct((NR, F), jnp.float32),
    )(cnt, u, z, b)


def _tc_c(cnt, u, z, w2, b2, w3):
    return pl.pallas_call(
        _body_c,
        grid=(GRID,),
        in_specs=[_cnt_spec(), _u_spec(), _row_spec(F), _full_spec((F, 128)),
                  _full_spec((1, 128)), _full_spec((128, F))],
        out_specs=_row_spec(F),
        out_shape=jax.ShapeDtypeStruct((NR, F), jnp.float32),
    )(cnt, u, z, w2, b2, w3)


def _tc_e(cnt, u, z, wmu, bmu, wls, bls):
    return pl.pallas_call(
        _body_e,
        grid=(GRID,),
        in_specs=[_cnt_spec(), _u_spec(), _row_spec(F), _full_spec((F, 32)),
                  _full_spec((1, 32)), _full_spec((F, 32)), _full_spec((1, 32))],
        out_specs=[_row_spec(32), _row_spec(32)],
        out_shape=[jax.ShapeDtypeStruct((N, 32), jnp.float32),
                   jax.ShapeDtypeStruct((N, 32), jnp.float32)],
    )(cnt, u, z, wmu, bmu, wls, bls)


def kernel(x, edge_index, W1, b1, W2, b2, W3, b3, Wmu, bmu, Wls, bls):
    src = edge_index[0].astype(jnp.int32)
    dst = edge_index[1].astype(jnp.int32)
    pad = E_PAD - E
    srcs = jnp.concatenate([src, jnp.zeros((pad,), jnp.int32)]).reshape(NW, NCH, CH)
    dsts = jnp.concatenate([dst, jnp.full((pad,), N, jnp.int32)]).reshape(NW, NCH, CH)

    zeros16 = jnp.zeros((NR, 16), jnp.float32)
    ones_rows = jnp.ones((CH, 16), jnp.float32)
    cnt = _deg_counts(dsts, zeros16, ones_rows)

    z1 = _tc_a(cnt, x, W1)
    u1 = _agg_pass(z1, srcs, dsts)
    z2 = _tc_b(cnt, u1, z1, b1.reshape(1, F))
    u2 = _agg_pass(z2, srcs, dsts)
    z3 = _tc_c(cnt, u2, z2, W2, b2.reshape(1, 128), W3)
    u3 = _agg_pass(z3, srcs, dsts)
    z4 = _tc_b(cnt, u3, z3, b3.reshape(1, F))
    u4 = _agg_pass(z4, srcs, dsts)
    mu, ls = _tc_e(cnt, u4, z4, Wmu, bmu.reshape(1, 32), Wls, bls.reshape(1, 32))
    return (mu, ls)


# speedup vs baseline: 14.0681x; 1.1040x over previous
import functools

import jax
import jax.numpy as jnp
from jax import lax
from jax.experimental import pallas as pl
from jax.experimental.pallas import tpu as pltpu
from jax.experimental.pallas import tpu_sc as plsc

N = 10000
NR = 10240
F = 64
E = 320000
CH = 128
NW = 32
NCH = 80
EPT = NCH * CH
E_PAD = NW * EPT
RPS = NR // 16
NBUF = 4

_MESH = plsc.VectorSubcoreMesh(
    core_axis_name="c", subcore_axis_name="s", num_cores=2, num_subcores=16
)
_SC_PARAMS = pltpu.CompilerParams(use_tc_tiling_on_sc=False)


@functools.partial(
    pl.kernel,
    out_type=jax.ShapeDtypeStruct((2, NR, 16), jnp.float32),
    mesh=_MESH,
    scratch_types=[
        pltpu.VMEM((NCH, CH), jnp.int32),
        pltpu.VMEM((CH, 16), jnp.float32),
        pltpu.VMEM_SHARED((NR, 16), jnp.float32),
    ],
    compiler_params=_SC_PARAMS,
)
def _deg_counts(dsts_hbm, zeros_hbm, ones_hbm, out_hbm, dst_v, ones_v, acc):
    c = lax.axis_index("c")
    s = lax.axis_index("s")
    w = c * 16 + s
    pltpu.sync_copy(zeros_hbm.at[pl.ds(s * RPS, RPS)], acc.at[pl.ds(s * RPS, RPS)])
    pltpu.sync_copy(dsts_hbm.at[w], dst_v)
    pltpu.sync_copy(ones_hbm, ones_v)
    plsc.subcore_barrier()

    @pl.loop(0, NCH)
    def _(j):
        pltpu.sync_copy(ones_v, acc.at[dst_v.at[j]], add=True)

    plsc.subcore_barrier()
    pltpu.sync_copy(acc.at[pl.ds(s * RPS, RPS)], out_hbm.at[c, pl.ds(s * RPS, RPS)])


@functools.partial(
    pl.kernel,
    out_type=jax.ShapeDtypeStruct((2, NR, F), jnp.float32),
    mesh=_MESH,
    scratch_types=[
        pltpu.VMEM((NCH, CH), jnp.int32),
        pltpu.VMEM((NCH, CH), jnp.int32),
        [pltpu.VMEM((CH, F), jnp.float32)] * NBUF,
        pltpu.VMEM_SHARED((NR, F), jnp.float32),
        [pltpu.SemaphoreType.DMA] * NBUF,
        [pltpu.SemaphoreType.DMA] * NBUF,
    ],
    compiler_params=_SC_PARAMS,
)
def _agg_pass(z_hbm, srcs_hbm, dsts_hbm, out_hbm,
              src_v, dst_v, rows, acc, gsems, ssems):
    c = lax.axis_index("c")
    s = lax.axis_index("s")
    w = c * 16 + s
    pltpu.sync_copy(z_hbm.at[pl.ds(s * RPS, RPS)], acc.at[pl.ds(s * RPS, RPS)])
    pltpu.sync_copy(srcs_hbm.at[w], src_v)
    pltpu.sync_copy(dsts_hbm.at[w], dst_v)
    plsc.subcore_barrier()

    for b in range(NBUF):
        pltpu.async_copy(z_hbm.at[src_v.at[b]], rows[b], gsems[b])

    @pl.loop(0, NCH, step=NBUF)
    def _(j):
        scat = []
        for b in range(NBUF):
            pltpu.make_async_copy(z_hbm.at[src_v.at[j + b]], rows[b], gsems[b]).wait()
            scat.append(pltpu.async_copy(rows[b], acc.at[dst_v.at[j + b]],
                                         ssems[b], add=True))
        for b in range(NBUF):
            scat[b].wait()

            @pl.when(j + NBUF + b < NCH)
            def _():
                pltpu.async_copy(z_hbm.at[src_v.at[j + NBUF + b]], rows[b], gsems[b])

    plsc.subcore_barrier()
    pltpu.sync_copy(acc.at[pl.ds(s * RPS, RPS)], out_hbm.at[c, pl.ds(s * RPS, RPS)])


BLK = 2048
GRID = NR // BLK

def _dinv(cnt):
    return lax.rsqrt(1.0 + cnt[0, :, :1] + cnt[1, :, :1])


def _body_a(cnt_ref, x_ref, w1_ref, z1_ref):
    d = _dinv(cnt_ref[...])
    t = jnp.dot(x_ref[...], w1_ref[...], preferred_element_type=jnp.float32)
    z1_ref[...] = d * t


def _body_b(cnt_ref, u_ref, z_ref, b_ref, out_ref):
    d = _dinv(cnt_ref[...])
    u = u_ref[...]
    h = jnp.maximum(d * (u[0] + u[1] - z_ref[...]) + b_ref[...], 0.0)
    out_ref[...] = d * h


def _body_c(cnt_ref, u_ref, z_ref, w2_ref, b2_ref, w3_ref, out_ref):
    d = _dinv(cnt_ref[...])
    u = u_ref[...]
    a2 = d * (u[0] + u[1] - z_ref[...])
    h2 = jnp.maximum(
        jnp.dot(a2, w2_ref[...], preferred_element_type=jnp.float32) + b2_ref[...], 0.0)
    t3 = jnp.dot(h2, w3_ref[...], preferred_element_type=jnp.float32)
    out_ref[...] = d * t3


def _body_e(cnt_ref, u_ref, z_ref, wmu_ref, bmu_ref, wls_ref, bls_ref,
            mu_ref, ls_ref):
    d = _dinv(cnt_ref[...])
    u = u_ref[...]
    a4 = d * (u[0] + u[1] - z_ref[...])
    mu_ref[...] = jnp.dot(a4, wmu_ref[...], preferred_element_type=jnp.float32) + bmu_ref[...]
    ls_ref[...] = jnp.dot(a4, wls_ref[...], preferred_element_type=jnp.float32) + bls_ref[...]


def _row_spec(width):
    return pl.BlockSpec((BLK, width), lambda i: (i, 0))


def _cnt_spec():
    return pl.BlockSpec((2, BLK, 16), lambda i: (0, i, 0))


def _u_spec():
    return pl.BlockSpec((2, BLK, F), lambda i: (0, i, 0))


def _full_spec(shape):
    nd = len(shape)
    return pl.BlockSpec(shape, lambda i: (0,) * nd)


def _tc_a(cnt, x, w1):
    return pl.pallas_call(
        _body_a,
        grid=(GRID,),
        in_specs=[_cnt_spec(), _row_spec(128), _full_spec((128, F))],
        out_specs=_row_spec(F),
        out_shape=jax.ShapeDtypeStruct((NR, F), jnp.float32),
    )(cnt, x, w1)


def _tc_b(cnt, u, z, b):
    return pl.pallas_call(
        _body_b,
        grid=(GRID,),
        in_specs=[_cnt_spec(), _u_spec(), _row_spec(F), _full_spec((1, F))],
        out_specs=_row_spec(F),
        out_shape=jax.ShapeDtypeStruct((NR, F), jnp.float32),
    )(cnt, u, z, b)


def _tc_c(cnt, u, z, w2, b2, w3):
    return pl.pallas_call(
        _body_c,
        grid=(GRID,),
        in_specs=[_cnt_spec(), _u_spec(), _row_spec(F), _full_spec((F, 128)),
                  _full_spec((1, 128)), _full_spec((128, F))],
        out_specs=_row_spec(F),
        out_shape=jax.ShapeDtypeStruct((NR, F), jnp.float32),
    )(cnt, u, z, w2, b2, w3)


def _tc_e(cnt, u, z, wmu, bmu, wls, bls):
    return pl.pallas_call(
        _body_e,
        grid=(GRID,),
        in_specs=[_cnt_spec(), _u_spec(), _row_spec(F), _full_spec((F, 32)),
                  _full_spec((1, 32)), _full_spec((F, 32)), _full_spec((1, 32))],
        out_specs=[_row_spec(32), _row_spec(32)],
        out_shape=[jax.ShapeDtypeStruct((N, 32), jnp.float32),
                   jax.ShapeDtypeStruct((N, 32), jnp.float32)],
    )(cnt, u, z, wmu, bmu, wls, bls)


def kernel(x, edge_index, W1, b1, W2, b2, W3, b3, Wmu, bmu, Wls, bls):
    src = edge_index[0].astype(jnp.int32)
    dst = edge_index[1].astype(jnp.int32)
    pad = E_PAD - E
    srcs = jnp.concatenate([src, jnp.zeros((pad,), jnp.int32)]).reshape(NW, NCH, CH)
    dsts = jnp.concatenate([dst, jnp.full((pad,), N, jnp.int32)]).reshape(NW, NCH, CH)

    zeros16 = jnp.zeros((NR, 16), jnp.float32)
    ones_rows = jnp.ones((CH, 16), jnp.float32)
    cnt = _deg_counts(dsts, zeros16, ones_rows)

    z1 = _tc_a(cnt, x, W1)
    u1 = _agg_pass(z1, srcs, dsts)
    z2 = _tc_b(cnt, u1, z1, b1.reshape(1, F))
    u2 = _agg_pass(z2, srcs, dsts)
    z3 = _tc_c(cnt, u2, z2, W2, b2.reshape(1, 128), W3)
    u3 = _agg_pass(z3, srcs, dsts)
    z4 = _tc_b(cnt, u3, z3, b3.reshape(1, F))
    u4 = _agg_pass(z4, srcs, dsts)
    mu, ls = _tc_e(cnt, u4, z4, Wmu, bmu.reshape(1, 32), Wls, bls.reshape(1, 32))
    return (mu, ls)


# speedup vs baseline: 14.1366x; 1.0049x over previous
import functools

import jax
import jax.numpy as jnp
from jax import lax
from jax.experimental import pallas as pl
from jax.experimental.pallas import tpu as pltpu
from jax.experimental.pallas import tpu_sc as plsc

N = 10000
NR = 10240
F = 64
E = 320000
CH = 128
NW = 32
NCH = 80
EPT = NCH * CH
E_PAD = NW * EPT
RPS = NR // 16
NBUF = 4

_MESH = plsc.VectorSubcoreMesh(
    core_axis_name="c", subcore_axis_name="s", num_cores=2, num_subcores=16
)
_SC_PARAMS = pltpu.CompilerParams(use_tc_tiling_on_sc=False)


@functools.partial(
    pl.kernel,
    out_type=jax.ShapeDtypeStruct((2, NR, 16), jnp.float32),
    mesh=_MESH,
    scratch_types=[
        pltpu.VMEM((NCH, CH), jnp.int32),
        pltpu.VMEM((CH, 16), jnp.float32),
        pltpu.VMEM_SHARED((NR, 16), jnp.float32),
    ],
    compiler_params=_SC_PARAMS,
)
def _deg_counts(dsts_hbm, zeros_hbm, ones_hbm, out_hbm, dst_v, ones_v, acc):
    c = lax.axis_index("c")
    s = lax.axis_index("s")
    w = c * 16 + s
    pltpu.sync_copy(zeros_hbm.at[pl.ds(s * RPS, RPS)], acc.at[pl.ds(s * RPS, RPS)])
    pltpu.sync_copy(dsts_hbm.at[w], dst_v)
    pltpu.sync_copy(ones_hbm, ones_v)
    plsc.subcore_barrier()

    @pl.loop(0, NCH)
    def _(j):
        pltpu.sync_copy(ones_v, acc.at[dst_v.at[j]], add=True)

    plsc.subcore_barrier()
    pltpu.sync_copy(acc.at[pl.ds(s * RPS, RPS)], out_hbm.at[c, pl.ds(s * RPS, RPS)])


@functools.partial(
    pl.kernel,
    out_type=jax.ShapeDtypeStruct((2, NR, F), jnp.float32),
    mesh=_MESH,
    scratch_types=[
        pltpu.VMEM((NCH, CH), jnp.int32),
        pltpu.VMEM((NCH, CH), jnp.int32),
        [pltpu.VMEM((CH, F), jnp.float32)] * NBUF,
        pltpu.VMEM_SHARED((NR, F), jnp.float32),
        [pltpu.SemaphoreType.DMA] * NBUF,
        [pltpu.SemaphoreType.DMA] * NBUF,
    ],
    compiler_params=_SC_PARAMS,
)
def _agg_pass(z_hbm, srcs_hbm, dsts_hbm, out_hbm,
              src_v, dst_v, rows, acc, gsems, ssems):
    c = lax.axis_index("c")
    s = lax.axis_index("s")
    w = c * 16 + s
    pltpu.sync_copy(z_hbm.at[pl.ds(s * RPS, RPS)], acc.at[pl.ds(s * RPS, RPS)])
    pltpu.sync_copy(srcs_hbm.at[w], src_v)
    pltpu.sync_copy(dsts_hbm.at[w], dst_v)
    plsc.subcore_barrier()

    for b in range(NBUF):
        pltpu.async_copy(z_hbm.at[src_v.at[b]], rows[b], gsems[b])

    @pl.loop(0, NCH, step=NBUF)
    def _(j):
        scat = []
        for b in range(NBUF):
            pltpu.make_async_copy(z_hbm.at[src_v.at[j + b]], rows[b], gsems[b]).wait()
        for b in range(NBUF):

            @pl.when(j + NBUF + b < NCH)
            def _():
                pltpu.async_copy(z_hbm.at[src_v.at[j + NBUF + b]], rows[b], gsems[b])

    plsc.subcore_barrier()
    pltpu.sync_copy(acc.at[pl.ds(s * RPS, RPS)], out_hbm.at[c, pl.ds(s * RPS, RPS)])


BLK = 2048
GRID = NR // BLK

def _dinv(cnt):
    return lax.rsqrt(1.0 + cnt[0, :, :1] + cnt[1, :, :1])


def _body_a(cnt_ref, x_ref, w1_ref, z1_ref):
    d = _dinv(cnt_ref[...])
    t = jnp.dot(x_ref[...], w1_ref[...], preferred_element_type=jnp.float32)
    z1_ref[...] = d * t


def _body_b(cnt_ref, u_ref, z_ref, b_ref, out_ref):
    d = _dinv(cnt_ref[...])
    u = u_ref[...]
    h = jnp.maximum(d * (u[0] + u[1] - z_ref[...]) + b_ref[...], 0.0)
    out_ref[...] = d * h


def _body_c(cnt_ref, u_ref, z_ref, w2_ref, b2_ref, w3_ref, out_ref):
    d = _dinv(cnt_ref[...])
    u = u_ref[...]
    a2 = d * (u[0] + u[1] - z_ref[...])
    h2 = jnp.maximum(
        jnp.dot(a2, w2_ref[...], preferred_element_type=jnp.float32) + b2_ref[...], 0.0)
    t3 = jnp.dot(h2, w3_ref[...], preferred_element_type=jnp.float32)
    out_ref[...] = d * t3


def _body_e(cnt_ref, u_ref, z_ref, wmu_ref, bmu_ref, wls_ref, bls_ref,
            mu_ref, ls_ref):
    d = _dinv(cnt_ref[...])
    u = u_ref[...]
    a4 = d * (u[0] + u[1] - z_ref[...])
    mu_ref[...] = jnp.dot(a4, wmu_ref[...], preferred_element_type=jnp.float32) + bmu_ref[...]
    ls_ref[...] = jnp.dot(a4, wls_ref[...], preferred_element_type=jnp.float32) + bls_ref[...]


def _row_spec(width):
    return pl.BlockSpec((BLK, width), lambda i: (i, 0))


def _cnt_spec():
    return pl.BlockSpec((2, BLK, 16), lambda i: (0, i, 0))


def _u_spec():
    return pl.BlockSpec((2, BLK, F), lambda i: (0, i, 0))


def _full_spec(shape):
    nd = len(shape)
    return pl.BlockSpec(shape, lambda i: (0,) * nd)


def _tc_a(cnt, x, w1):
    return pl.pallas_call(
        _body_a,
        grid=(GRID,),
        in_specs=[_cnt_spec(), _row_spec(128), _full_spec((128, F))],
        out_specs=_row_spec(F),
        out_shape=jax.ShapeDtypeStruct((NR, F), jnp.float32),
    )(cnt, x, w1)


def _tc_b(cnt, u, z, b):
    return pl.pallas_call(
        _body_b,
        grid=(GRID,),
        in_specs=[_cnt_spec(), _u_spec(), _row_spec(F), _full_spec((1, F))],
        out_specs=_row_spec(F),
        out_shape=jax.ShapeDtypeStruct((NR, F), jnp.float32),
    )(cnt, u, z, b)


def _tc_c(cnt, u, z, w2, b2, w3):
    return pl.pallas_call(
        _body_c,
        grid=(GRID,),
        in_specs=[_cnt_spec(), _u_spec(), _row_spec(F), _full_spec((F, 128)),
                  _full_spec((1, 128)), _full_spec((128, F))],
        out_specs=_row_spec(F),
        out_shape=jax.ShapeDtypeStruct((NR, F), jnp.float32),
    )(cnt, u, z, w2, b2, w3)


def _tc_e(cnt, u, z, wmu, bmu, wls, bls):
    return pl.pallas_call(
        _body_e,
        grid=(GRID,),
        in_specs=[_cnt_spec(), _u_spec(), _row_spec(F), _full_spec((F, 32)),
                  _full_spec((1, 32)), _full_spec((F, 32)), _full_spec((1, 32))],
        out_specs=[_row_spec(32), _row_spec(32)],
        out_shape=[jax.ShapeDtypeStruct((N, 32), jnp.float32),
                   jax.ShapeDtypeStruct((N, 32), jnp.float32)],
    )(cnt, u, z, wmu, bmu, wls, bls)


def kernel(x, edge_index, W1, b1, W2, b2, W3, b3, Wmu, bmu, Wls, bls):
    src = edge_index[0].astype(jnp.int32)
    dst = edge_index[1].astype(jnp.int32)
    pad = E_PAD - E
    srcs = jnp.concatenate([src, jnp.zeros((pad,), jnp.int32)]).reshape(NW, NCH, CH)
    dsts = jnp.concatenate([dst, jnp.full((pad,), N, jnp.int32)]).reshape(NW, NCH, CH)

    zeros16 = jnp.zeros((NR, 16), jnp.float32)
    ones_rows = jnp.ones((CH, 16), jnp.float32)
    cnt = _deg_counts(dsts, zeros16, ones_rows)

    z1 = _tc_a(cnt, x, W1)
    u1 = _agg_pass(z1, srcs, dsts)
    z2 = _tc_b(cnt, u1, z1, b1.reshape(1, F))
    u2 = _agg_pass(z2, srcs, dsts)
    z3 = _tc_c(cnt, u2, z2, W2, b2.reshape(1, 128), W3)
    u3 = _agg_pass(z3, srcs, dsts)
    z4 = _tc_b(cnt, u3, z3, b3.reshape(1, F))
    u4 = _agg_pass(z4, srcs, dsts)
    mu, ls = _tc_e(cnt, u4, z4, Wmu, bmu.reshape(1, 32), Wls, bls.reshape(1, 32))
    return (mu, ls)


# speedup vs baseline: 14.2916x; 1.0110x over previous
import functools

import jax
import jax.numpy as jnp
from jax import lax
from jax.experimental import pallas as pl
from jax.experimental.pallas import tpu as pltpu
from jax.experimental.pallas import tpu_sc as plsc

N = 10000
NR = 10240
F = 64
E = 320000
CH = 128
NW = 32
NCH = 80
EPT = NCH * CH
E_PAD = NW * EPT
RPS = NR // 16
NBUF = 8

_MESH = plsc.VectorSubcoreMesh(
    core_axis_name="c", subcore_axis_name="s", num_cores=2, num_subcores=16
)
_SC_PARAMS = pltpu.CompilerParams(use_tc_tiling_on_sc=False)


@functools.partial(
    pl.kernel,
    out_type=jax.ShapeDtypeStruct((2, NR, 16), jnp.float32),
    mesh=_MESH,
    scratch_types=[
        pltpu.VMEM((NCH, CH), jnp.int32),
        pltpu.VMEM((CH, 16), jnp.float32),
        pltpu.VMEM_SHARED((NR, 16), jnp.float32),
    ],
    compiler_params=_SC_PARAMS,
)
def _deg_counts(dsts_hbm, zeros_hbm, ones_hbm, out_hbm, dst_v, ones_v, acc):
    c = lax.axis_index("c")
    s = lax.axis_index("s")
    w = c * 16 + s
    pltpu.sync_copy(zeros_hbm.at[pl.ds(s * RPS, RPS)], acc.at[pl.ds(s * RPS, RPS)])
    pltpu.sync_copy(dsts_hbm.at[w], dst_v)
    pltpu.sync_copy(ones_hbm, ones_v)
    plsc.subcore_barrier()

    @pl.loop(0, NCH)
    def _(j):
        pltpu.sync_copy(ones_v, acc.at[dst_v.at[j]], add=True)

    plsc.subcore_barrier()
    pltpu.sync_copy(acc.at[pl.ds(s * RPS, RPS)], out_hbm.at[c, pl.ds(s * RPS, RPS)])


@functools.partial(
    pl.kernel,
    out_type=jax.ShapeDtypeStruct((2, NR, F), jnp.float32),
    mesh=_MESH,
    scratch_types=[
        pltpu.VMEM((NCH, CH), jnp.int32),
        pltpu.VMEM((NCH, CH), jnp.int32),
        [pltpu.VMEM((CH, F), jnp.float32)] * NBUF,
        pltpu.VMEM_SHARED((NR, F), jnp.float32),
        [pltpu.SemaphoreType.DMA] * NBUF,
        [pltpu.SemaphoreType.DMA] * NBUF,
    ],
    compiler_params=_SC_PARAMS,
)
def _agg_pass(z_hbm, srcs_hbm, dsts_hbm, out_hbm,
              src_v, dst_v, rows, acc, gsems, ssems):
    c = lax.axis_index("c")
    s = lax.axis_index("s")
    w = c * 16 + s
    pltpu.sync_copy(z_hbm.at[pl.ds(s * RPS, RPS)], acc.at[pl.ds(s * RPS, RPS)])
    pltpu.sync_copy(srcs_hbm.at[w], src_v)
    pltpu.sync_copy(dsts_hbm.at[w], dst_v)
    plsc.subcore_barrier()

    for b in range(NBUF):
        pltpu.async_copy(z_hbm.at[src_v.at[b]], rows[b], gsems[b])

    @pl.loop(0, NCH, step=NBUF)
    def _(j):
        scat = []
        for b in range(NBUF):
            pltpu.make_async_copy(z_hbm.at[src_v.at[j + b]], rows[b], gsems[b]).wait()
            scat.append(pltpu.async_copy(rows[b], acc.at[dst_v.at[j + b]],
                                         ssems[b], add=True))
        for b in range(NBUF):
            scat[b].wait()

            @pl.when(j + NBUF + b < NCH)
            def _():
                pltpu.async_copy(z_hbm.at[src_v.at[j + NBUF + b]], rows[b], gsems[b])

    plsc.subcore_barrier()
    pltpu.sync_copy(acc.at[pl.ds(s * RPS, RPS)], out_hbm.at[c, pl.ds(s * RPS, RPS)])


BLK = 2048
GRID = NR // BLK

def _dinv(cnt):
    return lax.rsqrt(1.0 + cnt[0, :, :1] + cnt[1, :, :1])


def _body_a(cnt_ref, x_ref, w1_ref, z1_ref):
    d = _dinv(cnt_ref[...])
    t = jnp.dot(x_ref[...], w1_ref[...], preferred_element_type=jnp.float32)
    z1_ref[...] = d * t


def _body_b(cnt_ref, u_ref, z_ref, b_ref, out_ref):
    d = _dinv(cnt_ref[...])
    u = u_ref[...]
    h = jnp.maximum(d * (u[0] + u[1] - z_ref[...]) + b_ref[...], 0.0)
    out_ref[...] = d * h


def _body_c(cnt_ref, u_ref, z_ref, w2_ref, b2_ref, w3_ref, out_ref):
    d = _dinv(cnt_ref[...])
    u = u_ref[...]
    a2 = d * (u[0] + u[1] - z_ref[...])
    h2 = jnp.maximum(
        jnp.dot(a2, w2_ref[...], preferred_element_type=jnp.float32) + b2_ref[...], 0.0)
    t3 = jnp.dot(h2, w3_ref[...], preferred_element_type=jnp.float32)
    out_ref[...] = d * t3


def _body_e(cnt_ref, u_ref, z_ref, wmu_ref, bmu_ref, wls_ref, bls_ref,
            mu_ref, ls_ref):
    d = _dinv(cnt_ref[...])
    u = u_ref[...]
    a4 = d * (u[0] + u[1] - z_ref[...])
    mu_ref[...] = jnp.dot(a4, wmu_ref[...], preferred_element_type=jnp.float32) + bmu_ref[...]
    ls_ref[...] = jnp.dot(a4, wls_ref[...], preferred_element_type=jnp.float32) + bls_ref[...]


def _row_spec(width):
    return pl.BlockSpec((BLK, width), lambda i: (i, 0))


def _cnt_spec():
    return pl.BlockSpec((2, BLK, 16), lambda i: (0, i, 0))


def _u_spec():
    return pl.BlockSpec((2, BLK, F), lambda i: (0, i, 0))


def _full_spec(shape):
    nd = len(shape)
    return pl.BlockSpec(shape, lambda i: (0,) * nd)


def _tc_a(cnt, x, w1):
    return pl.pallas_call(
        _body_a,
        grid=(GRID,),
        in_specs=[_cnt_spec(), _row_spec(128), _full_spec((128, F))],
        out_specs=_row_spec(F),
        out_shape=jax.ShapeDtypeStruct((NR, F), jnp.float32),
    )(cnt, x, w1)


def _tc_b(cnt, u, z, b):
    return pl.pallas_call(
        _body_b,
        grid=(GRID,),
        in_specs=[_cnt_spec(), _u_spec(), _row_spec(F), _full_spec((1, F))],
        out_specs=_row_spec(F),
        out_shape=jax.ShapeDtypeStruct((NR, F), jnp.float32),
    )(cnt, u, z, b)


def _tc_c(cnt, u, z, w2, b2, w3):
    return pl.pallas_call(
        _body_c,
        grid=(GRID,),
        in_specs=[_cnt_spec(), _u_spec(), _row_spec(F), _full_spec((F, 128)),
                  _full_spec((1, 128)), _full_spec((128, F))],
        out_specs=_row_spec(F),
        out_shape=jax.ShapeDtypeStruct((NR, F), jnp.float32),
    )(cnt, u, z, w2, b2, w3)


def _tc_e(cnt, u, z, wmu, bmu, wls, bls):
    return pl.pallas_call(
        _body_e,
        grid=(GRID,),
        in_specs=[_cnt_spec(), _u_spec(), _row_spec(F), _full_spec((F, 32)),
                  _full_spec((1, 32)), _full_spec((F, 32)), _full_spec((1, 32))],
        out_specs=[_row_spec(32), _row_spec(32)],
        out_shape=[jax.ShapeDtypeStruct((N, 32), jnp.float32),
                   jax.ShapeDtypeStruct((N, 32), jnp.float32)],
    )(cnt, u, z, wmu, bmu, wls, bls)


def kernel(x, edge_index, W1, b1, W2, b2, W3, b3, Wmu, bmu, Wls, bls):
    src = edge_index[0].astype(jnp.int32)
    dst = edge_index[1].astype(jnp.int32)
    pad = E_PAD - E
    srcs = jnp.concatenate([src, jnp.zeros((pad,), jnp.int32)]).reshape(NW, NCH, CH)
    dsts = jnp.concatenate([dst, jnp.full((pad,), N, jnp.int32)]).reshape(NW, NCH, CH)

    zeros16 = jnp.zeros((NR, 16), jnp.float32)
    ones_rows = jnp.ones((CH, 16), jnp.float32)
    cnt = _deg_counts(dsts, zeros16, ones_rows)

    z1 = _tc_a(cnt, x, W1)
    u1 = _agg_pass(z1, srcs, dsts)
    z2 = _tc_b(cnt, u1, z1, b1.reshape(1, F))
    u2 = _agg_pass(z2, srcs, dsts)
    z3 = _tc_c(cnt, u2, z2, W2, b2.reshape(1, 128), W3)
    u3 = _agg_pass(z3, srcs, dsts)
    z4 = _tc_b(cnt, u3, z3, b3.reshape(1, F))
    u4 = _agg_pass(z4, srcs, dsts)
    mu, ls = _tc_e(cnt, u4, z4, Wmu, bmu.reshape(1, 32), Wls, bls.reshape(1, 32))
    return (mu, ls)


# speedup vs baseline: 26.4591x; 1.8514x over previous
import functools

import jax
import jax.numpy as jnp
from jax import lax
from jax.experimental import pallas as pl
from jax.experimental.pallas import tpu as pltpu
from jax.experimental.pallas import tpu_sc as plsc

N = 10000
NR = 10240
F = 64
E = 320000
CH = 128
NW = 32
NCH = 80
EPT = NCH * CH
E_PAD = NW * EPT
RPS = NR // 16
NBUF = 4
FH = F // 2

_MESH = plsc.VectorSubcoreMesh(
    core_axis_name="c", subcore_axis_name="s", num_cores=2, num_subcores=16
)
_SC_PARAMS = pltpu.CompilerParams(use_tc_tiling_on_sc=False)


@functools.partial(
    pl.kernel,
    out_type=jax.ShapeDtypeStruct((2, NR, 16), jnp.float32),
    mesh=_MESH,
    scratch_types=[
        pltpu.VMEM((NCH, CH), jnp.int32),
        pltpu.VMEM((CH, 16), jnp.float32),
        pltpu.VMEM_SHARED((NR, 16), jnp.float32),
    ],
    compiler_params=_SC_PARAMS,
)
def _deg_counts(dsts_hbm, zeros_hbm, ones_hbm, out_hbm, dst_v, ones_v, acc):
    c = lax.axis_index("c")
    s = lax.axis_index("s")
    w = c * 16 + s
    pltpu.sync_copy(zeros_hbm.at[pl.ds(s * RPS, RPS)], acc.at[pl.ds(s * RPS, RPS)])
    pltpu.sync_copy(dsts_hbm.at[w], dst_v)
    pltpu.sync_copy(ones_hbm, ones_v)
    plsc.subcore_barrier()

    @pl.loop(0, NCH)
    def _(j):
        pltpu.sync_copy(ones_v, acc.at[dst_v.at[j]], add=True)

    plsc.subcore_barrier()
    pltpu.sync_copy(acc.at[pl.ds(s * RPS, RPS)], out_hbm.at[c, pl.ds(s * RPS, RPS)])


@functools.partial(
    pl.kernel,
    out_type=jax.ShapeDtypeStruct((2, NR, F), jnp.float32),
    mesh=_MESH,
    scratch_types=[
        pltpu.VMEM((NCH, CH), jnp.int32),
        pltpu.VMEM((NCH, CH), jnp.int32),
        [pltpu.VMEM((CH, FH), jnp.float32)] * NBUF,
        pltpu.VMEM_SHARED((NR, FH), jnp.float32),
        pltpu.VMEM_SHARED((NR, FH), jnp.float32),
        [pltpu.SemaphoreType.DMA] * NBUF,
        [pltpu.SemaphoreType.DMA] * NBUF,
    ],
    compiler_params=_SC_PARAMS,
)
def _agg_pass(z_hbm, srcs_hbm, dsts_hbm, out_hbm,
              src_v, dst_v, rows, acc, ztab, gsems, ssems):
    c = lax.axis_index("c")
    s = lax.axis_index("s")
    w = c * 16 + s
    pltpu.sync_copy(srcs_hbm.at[w], src_v)
    pltpu.sync_copy(dsts_hbm.at[w], dst_v)

    for p in range(2):
        pltpu.sync_copy(z_hbm.at[pl.ds(s * RPS, RPS), pl.ds(p * FH, FH)],
                        acc.at[pl.ds(s * RPS, RPS)])
        pltpu.sync_copy(z_hbm.at[pl.ds(s * RPS, RPS), pl.ds(p * FH, FH)],
                        ztab.at[pl.ds(s * RPS, RPS)])
        plsc.subcore_barrier()

        for b in range(NBUF):
            pltpu.async_copy(ztab.at[src_v.at[b]], rows[b], gsems[b])

        @pl.loop(0, NCH, step=NBUF)
        def _(j):
            scat = []
            for b in range(NBUF):
                pltpu.make_async_copy(ztab.at[src_v.at[j + b]], rows[b], gsems[b]).wait()
                scat.append(pltpu.async_copy(rows[b], acc.at[dst_v.at[j + b]],
                                             ssems[b], add=True))
            for b in range(NBUF):
                scat[b].wait()

                @pl.when(j + NBUF + b < NCH)
                def _():
                    pltpu.async_copy(ztab.at[src_v.at[j + NBUF + b]], rows[b], gsems[b])

        plsc.subcore_barrier()
        pltpu.sync_copy(acc.at[pl.ds(s * RPS, RPS)],
                        out_hbm.at[c, pl.ds(s * RPS, RPS), pl.ds(p * FH, FH)])
        plsc.subcore_barrier()


BLK = 2048
GRID = NR // BLK

def _dinv(cnt):
    return lax.rsqrt(1.0 + cnt[0, :, :1] + cnt[1, :, :1])


def _body_a(cnt_ref, x_ref, w1_ref, z1_ref):
    d = _dinv(cnt_ref[...])
    t = jnp.dot(x_ref[...], w1_ref[...], preferred_element_type=jnp.float32)
    z1_ref[...] = d * t


def _body_b(cnt_ref, u_ref, z_ref, b_ref, out_ref):
    d = _dinv(cnt_ref[...])
    u = u_ref[...]
    h = jnp.maximum(d * (u[0] + u[1] - z_ref[...]) + b_ref[...], 0.0)
    out_ref[...] = d * h


def _body_c(cnt_ref, u_ref, z_ref, w2_ref, b2_ref, w3_ref, out_ref):
    d = _dinv(cnt_ref[...])
    u = u_ref[...]
    a2 = d * (u[0] + u[1] - z_ref[...])
    h2 = jnp.maximum(
        jnp.dot(a2, w2_ref[...], preferred_element_type=jnp.float32) + b2_ref[...], 0.0)
    t3 = jnp.dot(h2, w3_ref[...], preferred_element_type=jnp.float32)
    out_ref[...] = d * t3


def _body_e(cnt_ref, u_ref, z_ref, wmu_ref, bmu_ref, wls_ref, bls_ref,
            mu_ref, ls_ref):
    d = _dinv(cnt_ref[...])
    u = u_ref[...]
    a4 = d * (u[0] + u[1] - z_ref[...])
    mu_ref[...] = jnp.dot(a4, wmu_ref[...], preferred_element_type=jnp.float32) + bmu_ref[...]
    ls_ref[...] = jnp.dot(a4, wls_ref[...], preferred_element_type=jnp.float32) + bls_ref[...]


def _row_spec(width):
    return pl.BlockSpec((BLK, width), lambda i: (i, 0))


def _cnt_spec():
    return pl.BlockSpec((2, BLK, 16), lambda i: (0, i, 0))


def _u_spec():
    return pl.BlockSpec((2, BLK, F), lambda i: (0, i, 0))


def _full_spec(shape):
    nd = len(shape)
    return pl.BlockSpec(shape, lambda i: (0,) * nd)


def _tc_a(cnt, x, w1):
    return pl.pallas_call(
        _body_a,
        grid=(GRID,),
        in_specs=[_cnt_spec(), _row_spec(128), _full_spec((128, F))],
        out_specs=_row_spec(F),
        out_shape=jax.ShapeDtypeStruct((NR, F), jnp.float32),
    )(cnt, x, w1)


def _tc_b(cnt, u, z, b):
    return pl.pallas_call(
        _body_b,
        grid=(GRID,),
        in_specs=[_cnt_spec(), _u_spec(), _row_spec(F), _full_spec((1, F))],
        out_specs=_row_spec(F),
        out_shape=jax.ShapeDtypeStruct((NR, F), jnp.float32),
    )(cnt, u, z, b)


def _tc_c(cnt, u, z, w2, b2, w3):
    return pl.pallas_call(
        _body_c,
        grid=(GRID,),
        in_specs=[_cnt_spec(), _u_spec(), _row_spec(F), _full_spec((F, 128)),
                  _full_spec((1, 128)), _full_spec((128, F))],
        out_specs=_row_spec(F),
        out_shape=jax.ShapeDtypeStruct((NR, F), jnp.float32),
    )(cnt, u, z, w2, b2, w3)


def _tc_e(cnt, u, z, wmu, bmu, wls, bls):
    return pl.pallas_call(
        _body_e,
        grid=(GRID,),
        in_specs=[_cnt_spec(), _u_spec(), _row_spec(F), _full_spec((F, 32)),
                  _full_spec((1, 32)), _full_spec((F, 32)), _full_spec((1, 32))],
        out_specs=[_row_spec(32), _row_spec(32)],
        out_shape=[jax.ShapeDtypeStruct((N, 32), jnp.float32),
                   jax.ShapeDtypeStruct((N, 32), jnp.float32)],
    )(cnt, u, z, wmu, bmu, wls, bls)


def kernel(x, edge_index, W1, b1, W2, b2, W3, b3, Wmu, bmu, Wls, bls):
    src = edge_index[0].astype(jnp.int32)
    dst = edge_index[1].astype(jnp.int32)
    pad = E_PAD - E
    srcs = jnp.concatenate([src, jnp.zeros((pad,), jnp.int32)]).reshape(NW, NCH, CH)
    dsts = jnp.concatenate([dst, jnp.full((pad,), N, jnp.int32)]).reshape(NW, NCH, CH)

    zeros16 = jnp.zeros((NR, 16), jnp.float32)
    ones_rows = jnp.ones((CH, 16), jnp.float32)
    cnt = _deg_counts(dsts, zeros16, ones_rows)

    z1 = _tc_a(cnt, x, W1)
    u1 = _agg_pass(z1, srcs, dsts)
    z2 = _tc_b(cnt, u1, z1, b1.reshape(1, F))
    u2 = _agg_pass(z2, srcs, dsts)
    z3 = _tc_c(cnt, u2, z2, W2, b2.reshape(1, 128), W3)
    u3 = _agg_pass(z3, srcs, dsts)
    z4 = _tc_b(cnt, u3, z3, b3.reshape(1, F))
    u4 = _agg_pass(z4, srcs, dsts)
    mu, ls = _tc_e(cnt, u4, z4, Wmu, bmu.reshape(1, 32), Wls, bls.reshape(1, 32))
    return (mu, ls)


# speedup vs baseline: 29.4884x; 1.1145x over previous
import functools

import jax
import jax.numpy as jnp
from jax import lax
from jax.experimental import pallas as pl
from jax.experimental.pallas import tpu as pltpu
from jax.experimental.pallas import tpu_sc as plsc

N = 10000
NR = 10240
F = 64
E = 320000
CH = 128
NCH = 160
NCHD = NCH // 2
E_PAD = 16 * NCH * CH
RPS = NR // 16
NBUF = 4
FH = F // 2

_MESH = plsc.VectorSubcoreMesh(
    core_axis_name="c", subcore_axis_name="s", num_cores=2, num_subcores=16
)
_SC_PARAMS = pltpu.CompilerParams(use_tc_tiling_on_sc=False)


@functools.partial(
    pl.kernel,
    out_type=jax.ShapeDtypeStruct((2, NR, 16), jnp.float32),
    mesh=_MESH,
    scratch_types=[
        pltpu.VMEM((NCHD, CH), jnp.int32),
        pltpu.VMEM((CH, 16), jnp.float32),
        pltpu.VMEM_SHARED((NR, 16), jnp.float32),
    ],
    compiler_params=_SC_PARAMS,
)
def _deg_counts(dsts_hbm, zeros_hbm, ones_hbm, out_hbm, dst_v, ones_v, acc):
    c = lax.axis_index("c")
    s = lax.axis_index("s")
    pltpu.sync_copy(zeros_hbm.at[pl.ds(s * RPS, RPS)], acc.at[pl.ds(s * RPS, RPS)])
    pltpu.sync_copy(dsts_hbm.at[s, pl.ds(c * NCHD, NCHD)], dst_v)
    pltpu.sync_copy(ones_hbm, ones_v)
    plsc.subcore_barrier()

    @pl.loop(0, NCHD)
    def _(j):
        pltpu.sync_copy(ones_v, acc.at[dst_v.at[j]], add=True)

    plsc.subcore_barrier()
    pltpu.sync_copy(acc.at[pl.ds(s * RPS, RPS)], out_hbm.at[c, pl.ds(s * RPS, RPS)])


@functools.partial(
    pl.kernel,
    out_type=jax.ShapeDtypeStruct((NR, F), jnp.float32),
    mesh=_MESH,
    scratch_types=[
        pltpu.VMEM((NCH, CH), jnp.int32),
        pltpu.VMEM((NCH, CH), jnp.int32),
        [pltpu.VMEM((CH, FH), jnp.float32)] * NBUF,
        pltpu.VMEM_SHARED((NR, FH), jnp.float32),
        pltpu.VMEM_SHARED((NR, FH), jnp.float32),
        [pltpu.SemaphoreType.DMA] * NBUF,
        [pltpu.SemaphoreType.DMA] * NBUF,
    ],
    compiler_params=_SC_PARAMS,
)
def _agg_pass(z_hbm, srcs_hbm, dsts_hbm, out_hbm,
              src_v, dst_v, rows, acc, ztab, gsems, ssems):
    c = lax.axis_index("c")
    s = lax.axis_index("s")
    pltpu.sync_copy(srcs_hbm.at[s], src_v)
    pltpu.sync_copy(dsts_hbm.at[s], dst_v)

    pltpu.sync_copy(z_hbm.at[pl.ds(s * RPS, RPS), pl.ds(c * FH, FH)],
                    acc.at[pl.ds(s * RPS, RPS)])
    pltpu.sync_copy(z_hbm.at[pl.ds(s * RPS, RPS), pl.ds(c * FH, FH)],
                    ztab.at[pl.ds(s * RPS, RPS)])
    plsc.subcore_barrier()

    for b in range(NBUF):
        pltpu.async_copy(ztab.at[src_v.at[b]], rows[b], gsems[b])

    @pl.loop(0, NCH, step=NBUF)
    def _(j):
        scat = []
        for b in range(NBUF):
            pltpu.make_async_copy(ztab.at[src_v.at[j + b]], rows[b], gsems[b]).wait()
            scat.append(pltpu.async_copy(rows[b], acc.at[dst_v.at[j + b]],
                                         ssems[b], add=True))
        for b in range(NBUF):
            scat[b].wait()

            @pl.when(j + NBUF + b < NCH)
            def _():
                pltpu.async_copy(ztab.at[src_v.at[j + NBUF + b]], rows[b], gsems[b])

    plsc.subcore_barrier()
    pltpu.sync_copy(acc.at[pl.ds(s * RPS, RPS)],
                    out_hbm.at[pl.ds(s * RPS, RPS), pl.ds(c * FH, FH)])


BLK = 2048
GRID = NR // BLK

def _dinv(cnt):
    return lax.rsqrt(1.0 + cnt[0, :, :1] + cnt[1, :, :1])


def _body_a(cnt_ref, x_ref, w1_ref, z1_ref):
    d = _dinv(cnt_ref[...])
    t = jnp.dot(x_ref[...], w1_ref[...], preferred_element_type=jnp.float32)
    z1_ref[...] = d * t


def _body_b(cnt_ref, u_ref, b_ref, out_ref):
    d = _dinv(cnt_ref[...])
    h = jnp.maximum(d * u_ref[...] + b_ref[...], 0.0)
    out_ref[...] = d * h


def _body_c(cnt_ref, u_ref, w2_ref, b2_ref, w3_ref, out_ref):
    d = _dinv(cnt_ref[...])
    a2 = d * u_ref[...]
    h2 = jnp.maximum(
        jnp.dot(a2, w2_ref[...], preferred_element_type=jnp.float32) + b2_ref[...], 0.0)
    t3 = jnp.dot(h2, w3_ref[...], preferred_element_type=jnp.float32)
    out_ref[...] = d * t3


def _body_e(cnt_ref, u_ref, wmu_ref, bmu_ref, wls_ref, bls_ref,
            mu_ref, ls_ref):
    d = _dinv(cnt_ref[...])
    a4 = d * u_ref[...]
    mu_ref[...] = jnp.dot(a4, wmu_ref[...], preferred_element_type=jnp.float32) + bmu_ref[...]
    ls_ref[...] = jnp.dot(a4, wls_ref[...], preferred_element_type=jnp.float32) + bls_ref[...]


def _row_spec(width):
    return pl.BlockSpec((BLK, width), lambda i: (i, 0))


def _cnt_spec():
    return pl.BlockSpec((2, BLK, 16), lambda i: (0, i, 0))


def _full_spec(shape):
    nd = len(shape)
    return pl.BlockSpec(shape, lambda i: (0,) * nd)


def _tc_a(cnt, x, w1):
    return pl.pallas_call(
        _body_a,
        grid=(GRID,),
        in_specs=[_cnt_spec(), _row_spec(128), _full_spec((128, F))],
        out_specs=_row_spec(F),
        out_shape=jax.ShapeDtypeStruct((NR, F), jnp.float32),
    )(cnt, x, w1)


def _tc_b(cnt, u, b):
    return pl.pallas_call(
        _body_b,
        grid=(GRID,),
        in_specs=[_cnt_spec(), _row_spec(F), _full_spec((1, F))],
        out_specs=_row_spec(F),
        out_shape=jax.ShapeDtypeStruct((NR, F), jnp.float32),
    )(cnt, u, b)


def _tc_c(cnt, u, w2, b2, w3):
    return pl.pallas_call(
        _body_c,
        grid=(GRID,),
        in_specs=[_cnt_spec(), _row_spec(F), _full_spec((F, 128)),
                  _full_spec((1, 128)), _full_spec((128, F))],
        out_specs=_row_spec(F),
        out_shape=jax.ShapeDtypeStruct((NR, F), jnp.float32),
    )(cnt, u, w2, b2, w3)


def _tc_e(cnt, u, wmu, bmu, wls, bls):
    return pl.pallas_call(
        _body_e,
        grid=(GRID,),
        in_specs=[_cnt_spec(), _row_spec(F), _full_spec((F, 32)),
                  _full_spec((1, 32)), _full_spec((F, 32)), _full_spec((1, 32))],
        out_specs=[_row_spec(32), _row_spec(32)],
        out_shape=[jax.ShapeDtypeStruct((N, 32), jnp.float32),
                   jax.ShapeDtypeStruct((N, 32), jnp.float32)],
    )(cnt, u, wmu, bmu, wls, bls)


def kernel(x, edge_index, W1, b1, W2, b2, W3, b3, Wmu, bmu, Wls, bls):
    src = edge_index[0].astype(jnp.int32)
    dst = edge_index[1].astype(jnp.int32)
    pad = E_PAD - E
    srcs = jnp.concatenate([src, jnp.zeros((pad,), jnp.int32)]).reshape(16, NCH, CH)
    dsts = jnp.concatenate([dst, jnp.full((pad,), N, jnp.int32)]).reshape(16, NCH, CH)

    zeros16 = jnp.zeros((NR, 16), jnp.float32)
    ones_rows = jnp.ones((CH, 16), jnp.float32)
    cnt = _deg_counts(dsts, zeros16, ones_rows)

    z1 = _tc_a(cnt, x, W1)
    u1 = _agg_pass(z1, srcs, dsts)
    z2 = _tc_b(cnt, u1, b1.reshape(1, F))
    u2 = _agg_pass(z2, srcs, dsts)
    z3 = _tc_c(cnt, u2, W2, b2.reshape(1, 128), W3)
    u3 = _agg_pass(z3, srcs, dsts)
    z4 = _tc_b(cnt, u3, b3.reshape(1, F))
    u4 = _agg_pass(z4, srcs, dsts)
    mu, ls = _tc_e(cnt, u4, Wmu, bmu.reshape(1, 32), Wls, bls.reshape(1, 32))
    return (mu, ls)


# speedup vs baseline: 30.6963x; 1.0410x over previous
import functools

import jax
import jax.numpy as jnp
from jax import lax
from jax.experimental import pallas as pl
from jax.experimental.pallas import tpu as pltpu
from jax.experimental.pallas import tpu_sc as plsc

N = 10000
NR = 10240
F = 64
E = 320000
CH = 128
NCH = 160
NCHD = NCH // 2
E_PAD = 16 * NCH * CH
RPS = NR // 16
NBUF = 4
FH = F // 2

_MESH = plsc.VectorSubcoreMesh(
    core_axis_name="c", subcore_axis_name="s", num_cores=2, num_subcores=16
)
_SC_PARAMS = pltpu.CompilerParams(use_tc_tiling_on_sc=False)


@functools.partial(
    pl.kernel,
    out_type=jax.ShapeDtypeStruct((2, NR, 16), jnp.float32),
    mesh=_MESH,
    scratch_types=[
        pltpu.VMEM((NCHD, CH), jnp.int32),
        pltpu.VMEM((CH, 16), jnp.float32),
        pltpu.VMEM_SHARED((NR, 16), jnp.float32),
    ],
    compiler_params=_SC_PARAMS,
)
def _deg_counts(dsts_hbm, zeros_hbm, ones_hbm, out_hbm, dst_v, ones_v, acc):
    c = lax.axis_index("c")
    s = lax.axis_index("s")
    pltpu.sync_copy(zeros_hbm.at[pl.ds(s * RPS, RPS)], acc.at[pl.ds(s * RPS, RPS)])
    pltpu.sync_copy(dsts_hbm.at[s, pl.ds(c * NCHD, NCHD)], dst_v)
    pltpu.sync_copy(ones_hbm, ones_v)
    plsc.subcore_barrier()

    @pl.loop(0, NCHD)
    def _(j):
        pltpu.sync_copy(ones_v, acc.at[dst_v.at[j]], add=True)

    plsc.subcore_barrier()
    pltpu.sync_copy(acc.at[pl.ds(s * RPS, RPS)], out_hbm.at[c, pl.ds(s * RPS, RPS)])


@functools.partial(
    pl.kernel,
    out_type=jax.ShapeDtypeStruct((NR, F), jnp.float32),
    mesh=_MESH,
    scratch_types=[
        pltpu.VMEM((NCH, CH), jnp.int32),
        pltpu.VMEM((NCH, CH), jnp.int32),
        [pltpu.VMEM((CH, FH), jnp.float32)] * NBUF,
        pltpu.VMEM_SHARED((NR, FH), jnp.float32),
        pltpu.VMEM_SHARED((NR, FH), jnp.float32),
        [pltpu.SemaphoreType.DMA] * NBUF,
        [pltpu.SemaphoreType.DMA] * NBUF,
    ],
    compiler_params=_SC_PARAMS,
)
def _agg_pass(z_hbm, srcs_hbm, dsts_hbm, out_hbm,
              src_v, dst_v, rows, acc, ztab, gsems, ssems):
    c = lax.axis_index("c")
    s = lax.axis_index("s")
    pro = [
        pltpu.async_copy(srcs_hbm.at[s], src_v, ssems[0]),
        pltpu.async_copy(dsts_hbm.at[s], dst_v, ssems[1]),
        pltpu.async_copy(z_hbm.at[pl.ds(s * RPS, RPS), pl.ds(c * FH, FH)],
                         acc.at[pl.ds(s * RPS, RPS)], ssems[2]),
        pltpu.async_copy(z_hbm.at[pl.ds(s * RPS, RPS), pl.ds(c * FH, FH)],
                         ztab.at[pl.ds(s * RPS, RPS)], ssems[3]),
    ]
    for cp in pro:
        cp.wait()
    plsc.subcore_barrier()

    for b in range(NBUF):
        pltpu.async_copy(ztab.at[src_v.at[b]], rows[b], gsems[b])

    @pl.loop(0, NCH, step=NBUF)
    def _(j):
        scat = []
        for b in range(NBUF):
            pltpu.make_async_copy(ztab.at[src_v.at[j + b]], rows[b], gsems[b]).wait()
            scat.append(pltpu.async_copy(rows[b], acc.at[dst_v.at[j + b]],
                                         ssems[b], add=True))
        for b in range(NBUF):
            scat[b].wait()

            @pl.when(j + NBUF + b < NCH)
            def _():
                pltpu.async_copy(ztab.at[src_v.at[j + NBUF + b]], rows[b], gsems[b])

    plsc.subcore_barrier()
    pltpu.sync_copy(acc.at[pl.ds(s * RPS, RPS)],
                    out_hbm.at[pl.ds(s * RPS, RPS), pl.ds(c * FH, FH)])


BLK = 2048
GRID = NR // BLK

def _dinv(cnt):
    return lax.rsqrt(1.0 + cnt[0, :, :1] + cnt[1, :, :1])


def _body_a(cnt_ref, x_ref, w1_ref, z1_ref):
    d = _dinv(cnt_ref[...])
    t = jnp.dot(x_ref[...], w1_ref[...], preferred_element_type=jnp.float32)
    z1_ref[...] = d * t


def _body_b(cnt_ref, u_ref, b_ref, out_ref):
    d = _dinv(cnt_ref[...])
    h = jnp.maximum(d * u_ref[...] + b_ref[...], 0.0)
    out_ref[...] = d * h


def _body_c(cnt_ref, u_ref, w2_ref, b2_ref, w3_ref, out_ref):
    d = _dinv(cnt_ref[...])
    a2 = d * u_ref[...]
    h2 = jnp.maximum(
        jnp.dot(a2, w2_ref[...], preferred_element_type=jnp.float32) + b2_ref[...], 0.0)
    t3 = jnp.dot(h2, w3_ref[...], preferred_element_type=jnp.float32)
    out_ref[...] = d * t3


def _body_e(cnt_ref, u_ref, wmu_ref, bmu_ref, wls_ref, bls_ref,
            mu_ref, ls_ref):
    d = _dinv(cnt_ref[...])
    a4 = d * u_ref[...]
    mu_ref[...] = jnp.dot(a4, wmu_ref[...], preferred_element_type=jnp.float32) + bmu_ref[...]
    ls_ref[...] = jnp.dot(a4, wls_ref[...], preferred_element_type=jnp.float32) + bls_ref[...]


def _row_spec(width):
    return pl.BlockSpec((BLK, width), lambda i: (i, 0))


def _cnt_spec():
    return pl.BlockSpec((2, BLK, 16), lambda i: (0, i, 0))


def _full_spec(shape):
    nd = len(shape)
    return pl.BlockSpec(shape, lambda i: (0,) * nd)


def _tc_a(cnt, x, w1):
    return pl.pallas_call(
        _body_a,
        grid=(GRID,),
        in_specs=[_cnt_spec(), _row_spec(128), _full_spec((128, F))],
        out_specs=_row_spec(F),
        out_shape=jax.ShapeDtypeStruct((NR, F), jnp.float32),
    )(cnt, x, w1)


def _tc_b(cnt, u, b):
    return pl.pallas_call(
        _body_b,
        grid=(GRID,),
        in_specs=[_cnt_spec(), _row_spec(F), _full_spec((1, F))],
        out_specs=_row_spec(F),
        out_shape=jax.ShapeDtypeStruct((NR, F), jnp.float32),
    )(cnt, u, b)


def _tc_c(cnt, u, w2, b2, w3):
    return pl.pallas_call(
        _body_c,
        grid=(GRID,),
        in_specs=[_cnt_spec(), _row_spec(F), _full_spec((F, 128)),
                  _full_spec((1, 128)), _full_spec((128, F))],
        out_specs=_row_spec(F),
        out_shape=jax.ShapeDtypeStruct((NR, F), jnp.float32),
    )(cnt, u, w2, b2, w3)


def _tc_e(cnt, u, wmu, bmu, wls, bls):
    return pl.pallas_call(
        _body_e,
        grid=(GRID,),
        in_specs=[_cnt_spec(), _row_spec(F), _full_spec((F, 32)),
                  _full_spec((1, 32)), _full_spec((F, 32)), _full_spec((1, 32))],
        out_specs=[_row_spec(32), _row_spec(32)],
        out_shape=[jax.ShapeDtypeStruct((N, 32), jnp.float32),
                   jax.ShapeDtypeStruct((N, 32), jnp.float32)],
    )(cnt, u, wmu, bmu, wls, bls)


def kernel(x, edge_index, W1, b1, W2, b2, W3, b3, Wmu, bmu, Wls, bls):
    src = edge_index[0].astype(jnp.int32)
    dst = edge_index[1].astype(jnp.int32)
    pad = E_PAD - E
    srcs = jnp.concatenate([src, jnp.zeros((pad,), jnp.int32)]).reshape(16, NCH, CH)
    dsts = jnp.concatenate([dst, jnp.full((pad,), N, jnp.int32)]).reshape(16, NCH, CH)

    zeros16 = jnp.zeros((NR, 16), jnp.float32)
    ones_rows = jnp.ones((CH, 16), jnp.float32)
    cnt = _deg_counts(dsts, zeros16, ones_rows)

    z1 = _tc_a(cnt, x, W1)
    u1 = _agg_pass(z1, srcs, dsts)
    z2 = _tc_b(cnt, u1, b1.reshape(1, F))
    u2 = _agg_pass(z2, srcs, dsts)
    z3 = _tc_c(cnt, u2, W2, b2.reshape(1, 128), W3)
    u3 = _agg_pass(z3, srcs, dsts)
    z4 = _tc_b(cnt, u3, b3.reshape(1, F))
    u4 = _agg_pass(z4, srcs, dsts)
    mu, ls = _tc_e(cnt, u4, Wmu, bmu.reshape(1, 32), Wls, bls.reshape(1, 32))
    return (mu, ls)


# speedup vs baseline: 31.4669x; 1.0251x over previous
import functools

import jax
import jax.numpy as jnp
from jax import lax
from jax.experimental import pallas as pl
from jax.experimental.pallas import tpu as pltpu
from jax.experimental.pallas import tpu_sc as plsc

N = 10000
NR = 10240
F = 64
E = 320000
CH = 128
NCH = 160
NCHD = NCH // 2
E_PAD = 16 * NCH * CH
RPS = NR // 16
NBUF = 8
FH = F // 2

_MESH = plsc.VectorSubcoreMesh(
    core_axis_name="c", subcore_axis_name="s", num_cores=2, num_subcores=16
)
_SC_PARAMS = pltpu.CompilerParams(use_tc_tiling_on_sc=False)


@functools.partial(
    pl.kernel,
    out_type=jax.ShapeDtypeStruct((2, NR, 16), jnp.float32),
    mesh=_MESH,
    scratch_types=[
        pltpu.VMEM((NCHD, CH), jnp.int32),
        pltpu.VMEM((CH, 16), jnp.float32),
        pltpu.VMEM_SHARED((NR, 16), jnp.float32),
    ],
    compiler_params=_SC_PARAMS,
)
def _deg_counts(dsts_hbm, zeros_hbm, ones_hbm, out_hbm, dst_v, ones_v, acc):
    c = lax.axis_index("c")
    s = lax.axis_index("s")
    pltpu.sync_copy(zeros_hbm.at[pl.ds(s * RPS, RPS)], acc.at[pl.ds(s * RPS, RPS)])
    pltpu.sync_copy(dsts_hbm.at[s, pl.ds(c * NCHD, NCHD)], dst_v)
    pltpu.sync_copy(ones_hbm, ones_v)
    plsc.subcore_barrier()

    @pl.loop(0, NCHD)
    def _(j):
        pltpu.sync_copy(ones_v, acc.at[dst_v.at[j]], add=True)

    plsc.subcore_barrier()
    pltpu.sync_copy(acc.at[pl.ds(s * RPS, RPS)], out_hbm.at[c, pl.ds(s * RPS, RPS)])


@functools.partial(
    pl.kernel,
    out_type=jax.ShapeDtypeStruct((NR, F), jnp.float32),
    mesh=_MESH,
    scratch_types=[
        pltpu.VMEM((NCH, CH), jnp.int32),
        pltpu.VMEM((NCH, CH), jnp.int32),
        [pltpu.VMEM((CH, FH), jnp.float32)] * NBUF,
        pltpu.VMEM_SHARED((NR, FH), jnp.float32),
        pltpu.VMEM_SHARED((NR, FH), jnp.float32),
        [pltpu.SemaphoreType.DMA] * NBUF,
        [pltpu.SemaphoreType.DMA] * NBUF,
    ],
    compiler_params=_SC_PARAMS,
)
def _agg_pass(z_hbm, srcs_hbm, dsts_hbm, out_hbm,
              src_v, dst_v, rows, acc, ztab, gsems, ssems):
    c = lax.axis_index("c")
    s = lax.axis_index("s")
    pro = [
        pltpu.async_copy(srcs_hbm.at[s], src_v, ssems[0]),
        pltpu.async_copy(dsts_hbm.at[s], dst_v, ssems[1]),
        pltpu.async_copy(z_hbm.at[pl.ds(s * RPS, RPS), pl.ds(c * FH, FH)],
                         acc.at[pl.ds(s * RPS, RPS)], ssems[2]),
        pltpu.async_copy(z_hbm.at[pl.ds(s * RPS, RPS), pl.ds(c * FH, FH)],
                         ztab.at[pl.ds(s * RPS, RPS)], ssems[3]),
    ]
    for cp in pro:
        cp.wait()
    plsc.subcore_barrier()

    for b in range(NBUF):
        pltpu.async_copy(ztab.at[src_v.at[b]], rows[b], gsems[b])

    @pl.loop(0, NCH, step=NBUF)
    def _(j):
        scat = []
        for b in range(NBUF):
            pltpu.make_async_copy(ztab.at[src_v.at[j + b]], rows[b], gsems[b]).wait()
            scat.append(pltpu.async_copy(rows[b], acc.at[dst_v.at[j + b]],
                                         ssems[b], add=True))
        for b in range(NBUF):
            scat[b].wait()

            @pl.when(j + NBUF + b < NCH)
            def _():
                pltpu.async_copy(ztab.at[src_v.at[j + NBUF + b]], rows[b], gsems[b])

    plsc.subcore_barrier()
    pltpu.sync_copy(acc.at[pl.ds(s * RPS, RPS)],
                    out_hbm.at[pl.ds(s * RPS, RPS), pl.ds(c * FH, FH)])


BLK = 2048
GRID = NR // BLK

def _dinv(cnt):
    return lax.rsqrt(1.0 + cnt[0, :, :1] + cnt[1, :, :1])


def _body_a(cnt_ref, x_ref, w1_ref, z1_ref):
    d = _dinv(cnt_ref[...])
    t = jnp.dot(x_ref[...], w1_ref[...], preferred_element_type=jnp.float32)
    z1_ref[...] = d * t


def _body_b(cnt_ref, u_ref, b_ref, out_ref):
    d = _dinv(cnt_ref[...])
    h = jnp.maximum(d * u_ref[...] + b_ref[...], 0.0)
    out_ref[...] = d * h


def _body_c(cnt_ref, u_ref, w2_ref, b2_ref, w3_ref, out_ref):
    d = _dinv(cnt_ref[...])
    a2 = d * u_ref[...]
    h2 = jnp.maximum(
        jnp.dot(a2, w2_ref[...], preferred_element_type=jnp.float32) + b2_ref[...], 0.0)
    t3 = jnp.dot(h2, w3_ref[...], preferred_element_type=jnp.float32)
    out_ref[...] = d * t3


def _body_e(cnt_ref, u_ref, wmu_ref, bmu_ref, wls_ref, bls_ref,
            mu_ref, ls_ref):
    d = _dinv(cnt_ref[...])
    a4 = d * u_ref[...]
    mu_ref[...] = jnp.dot(a4, wmu_ref[...], preferred_element_type=jnp.float32) + bmu_ref[...]
    ls_ref[...] = jnp.dot(a4, wls_ref[...], preferred_element_type=jnp.float32) + bls_ref[...]


def _row_spec(width):
    return pl.BlockSpec((BLK, width), lambda i: (i, 0))


def _cnt_spec():
    return pl.BlockSpec((2, BLK, 16), lambda i: (0, i, 0))


def _full_spec(shape):
    nd = len(shape)
    return pl.BlockSpec(shape, lambda i: (0,) * nd)


def _tc_a(cnt, x, w1):
    return pl.pallas_call(
        _body_a,
        grid=(GRID,),
        in_specs=[_cnt_spec(), _row_spec(128), _full_spec((128, F))],
        out_specs=_row_spec(F),
        out_shape=jax.ShapeDtypeStruct((NR, F), jnp.float32),
    )(cnt, x, w1)


def _tc_b(cnt, u, b):
    return pl.pallas_call(
        _body_b,
        grid=(GRID,),
        in_specs=[_cnt_spec(), _row_spec(F), _full_spec((1, F))],
        out_specs=_row_spec(F),
        out_shape=jax.ShapeDtypeStruct((NR, F), jnp.float32),
    )(cnt, u, b)


def _tc_c(cnt, u, w2, b2, w3):
    return pl.pallas_call(
        _body_c,
        grid=(GRID,),
        in_specs=[_cnt_spec(), _row_spec(F), _full_spec((F, 128)),
                  _full_spec((1, 128)), _full_spec((128, F))],
        out_specs=_row_spec(F),
        out_shape=jax.ShapeDtypeStruct((NR, F), jnp.float32),
    )(cnt, u, w2, b2, w3)


def _tc_e(cnt, u, wmu, bmu, wls, bls):
    return pl.pallas_call(
        _body_e,
        grid=(GRID,),
        in_specs=[_cnt_spec(), _row_spec(F), _full_spec((F, 32)),
                  _full_spec((1, 32)), _full_spec((F, 32)), _full_spec((1, 32))],
        out_specs=[_row_spec(32), _row_spec(32)],
        out_shape=[jax.ShapeDtypeStruct((N, 32), jnp.float32),
                   jax.ShapeDtypeStruct((N, 32), jnp.float32)],
    )(cnt, u, wmu, bmu, wls, bls)


def kernel(x, edge_index, W1, b1, W2, b2, W3, b3, Wmu, bmu, Wls, bls):
    src = edge_index[0].astype(jnp.int32)
    dst = edge_index[1].astype(jnp.int32)
    pad = E_PAD - E
    srcs = jnp.concatenate([src, jnp.zeros((pad,), jnp.int32)]).reshape(16, NCH, CH)
    dsts = jnp.concatenate([dst, jnp.full((pad,), N, jnp.int32)]).reshape(16, NCH, CH)

    zeros16 = jnp.zeros((NR, 16), jnp.float32)
    ones_rows = jnp.ones((CH, 16), jnp.float32)
    cnt = _deg_counts(dsts, zeros16, ones_rows)

    z1 = _tc_a(cnt, x, W1)
    u1 = _agg_pass(z1, srcs, dsts)
    z2 = _tc_b(cnt, u1, b1.reshape(1, F))
    u2 = _agg_pass(z2, srcs, dsts)
    z3 = _tc_c(cnt, u2, W2, b2.reshape(1, 128), W3)
    u3 = _agg_pass(z3, srcs, dsts)
    z4 = _tc_b(cnt, u3, b3.reshape(1, F))
    u4 = _agg_pass(z4, srcs, dsts)
    mu, ls = _tc_e(cnt, u4, Wmu, bmu.reshape(1, 32), Wls, bls.reshape(1, 32))
    return (mu, ls)


# speedup vs baseline: 32.1995x; 1.0233x over previous
import functools

import jax
import jax.numpy as jnp
from jax import lax
from jax.experimental import pallas as pl
from jax.experimental.pallas import tpu as pltpu
from jax.experimental.pallas import tpu_sc as plsc

N = 10000
NR = 10240
F = 64
E = 320000
CH = 128
NCH = 160
NCHD = NCH // 2
E_PAD = 16 * NCH * CH
RPS = NR // 16
NBUF = 8
FH = F // 2

_MESH = plsc.VectorSubcoreMesh(
    core_axis_name="c", subcore_axis_name="s", num_cores=2, num_subcores=16
)
_SC_PARAMS = pltpu.CompilerParams(use_tc_tiling_on_sc=False)


@functools.partial(
    pl.kernel,
    out_type=jax.ShapeDtypeStruct((2, NR, 16), jnp.float32),
    mesh=_MESH,
    scratch_types=[
        pltpu.VMEM((NCHD, CH), jnp.int32),
        pltpu.VMEM((CH, 16), jnp.float32),
        pltpu.VMEM_SHARED((NR, 16), jnp.float32),
    ],
    compiler_params=_SC_PARAMS,
)
def _deg_counts(dsts_hbm, zeros_hbm, ones_hbm, out_hbm, dst_v, ones_v, acc):
    c = lax.axis_index("c")
    s = lax.axis_index("s")
    pltpu.sync_copy(zeros_hbm.at[pl.ds(s * RPS, RPS)], acc.at[pl.ds(s * RPS, RPS)])
    pltpu.sync_copy(dsts_hbm.at[s, pl.ds(c * NCHD, NCHD)], dst_v)
    pltpu.sync_copy(ones_hbm, ones_v)
    plsc.subcore_barrier()

    @pl.loop(0, NCHD)
    def _(j):
        pltpu.sync_copy(ones_v, acc.at[dst_v.at[j]], add=True)

    plsc.subcore_barrier()
    pltpu.sync_copy(acc.at[pl.ds(s * RPS, RPS)], out_hbm.at[c, pl.ds(s * RPS, RPS)])


@functools.partial(
    pl.kernel,
    out_type=jax.ShapeDtypeStruct((NR, F), jnp.float32),
    mesh=_MESH,
    scratch_types=[
        pltpu.VMEM((NCH, CH), jnp.int32),
        pltpu.VMEM((NCH, CH), jnp.int32),
        [pltpu.VMEM((CH, FH), jnp.float32)] * NBUF,
        pltpu.VMEM_SHARED((NR, FH), jnp.float32),
        pltpu.VMEM_SHARED((NR, FH), jnp.float32),
        [pltpu.SemaphoreType.DMA] * NBUF,
        [pltpu.SemaphoreType.DMA] * NBUF,
    ],
    compiler_params=_SC_PARAMS,
)
def _agg_pass(z_hbm, srcs_hbm, dsts_hbm, out_hbm,
              src_v, dst_v, rows, acc, ztab, gsems, ssems):
    c = lax.axis_index("c")
    s = lax.axis_index("s")
    pro = [
        pltpu.async_copy(srcs_hbm.at[s], src_v, ssems[0]),
        pltpu.async_copy(dsts_hbm.at[s], dst_v, ssems[1]),
        pltpu.async_copy(z_hbm.at[pl.ds(s * RPS, RPS), pl.ds(c * FH, FH)],
                         acc.at[pl.ds(s * RPS, RPS)], ssems[2]),
        pltpu.async_copy(z_hbm.at[pl.ds(s * RPS, RPS), pl.ds(c * FH, FH)],
                         ztab.at[pl.ds(s * RPS, RPS)], ssems[3]),
    ]
    for cp in pro:
        cp.wait()
    plsc.subcore_barrier()

    for b in range(NBUF):
        pltpu.async_copy(ztab.at[src_v.at[b]], rows[b], gsems[b])

    @pl.loop(0, NCH, step=NBUF)
    def _(j):
        scat = []
        for b in range(NBUF):
            pltpu.make_async_copy(ztab.at[src_v.at[j + b]], rows[b], gsems[b]).wait()
            scat.append(pltpu.async_copy(rows[b], acc.at[dst_v.at[j + b]],
                                         ssems[b], add=True))
        for b in range(NBUF):
            scat[b].wait()

            @pl.when(j + NBUF + b < NCH)
            def _():
                pltpu.async_copy(ztab.at[src_v.at[j + NBUF + b]], rows[b], gsems[b])

    plsc.subcore_barrier()
    pltpu.sync_copy(acc.at[pl.ds(s * RPS, RPS)],
                    out_hbm.at[pl.ds(s * RPS, RPS), pl.ds(c * FH, FH)])


BLK = NR
GRID = NR // BLK

def _dinv(cnt):
    return lax.rsqrt(1.0 + cnt[0, :, :1] + cnt[1, :, :1])


def _body_a(cnt_ref, x_ref, w1_ref, z1_ref):
    d = _dinv(cnt_ref[...])
    t = jnp.dot(x_ref[...], w1_ref[...], preferred_element_type=jnp.float32)
    z1_ref[...] = d * t


def _body_b(cnt_ref, u_ref, b_ref, out_ref):
    d = _dinv(cnt_ref[...])
    h = jnp.maximum(d * u_ref[...] + b_ref[...], 0.0)
    out_ref[...] = d * h


def _body_c(cnt_ref, u_ref, w2_ref, b2_ref, w3_ref, out_ref):
    d = _dinv(cnt_ref[...])
    a2 = d * u_ref[...]
    h2 = jnp.maximum(
        jnp.dot(a2, w2_ref[...], preferred_element_type=jnp.float32) + b2_ref[...], 0.0)
    t3 = jnp.dot(h2, w3_ref[...], preferred_element_type=jnp.float32)
    out_ref[...] = d * t3


def _body_e(cnt_ref, u_ref, wmu_ref, bmu_ref, wls_ref, bls_ref,
            mu_ref, ls_ref):
    d = _dinv(cnt_ref[...])
    a4 = d * u_ref[...]
    mu_ref[...] = jnp.dot(a4, wmu_ref[...], preferred_element_type=jnp.float32) + bmu_ref[...]
    ls_ref[...] = jnp.dot(a4, wls_ref[...], preferred_element_type=jnp.float32) + bls_ref[...]


def _row_spec(width):
    return pl.BlockSpec((BLK, width), lambda i: (i, 0))


def _cnt_spec():
    return pl.BlockSpec((2, BLK, 16), lambda i: (0, i, 0))


def _full_spec(shape):
    nd = len(shape)
    return pl.BlockSpec(shape, lambda i: (0,) * nd)


def _tc_a(cnt, x, w1):
    return pl.pallas_call(
        _body_a,
        grid=(GRID,),
        in_specs=[_cnt_spec(), _row_spec(128), _full_spec((128, F))],
        out_specs=_row_spec(F),
        out_shape=jax.ShapeDtypeStruct((NR, F), jnp.float32),
    )(cnt, x, w1)


def _tc_b(cnt, u, b):
    return pl.pallas_call(
        _body_b,
        grid=(GRID,),
        in_specs=[_cnt_spec(), _row_spec(F), _full_spec((1, F))],
        out_specs=_row_spec(F),
        out_shape=jax.ShapeDtypeStruct((NR, F), jnp.float32),
    )(cnt, u, b)


def _tc_c(cnt, u, w2, b2, w3):
    return pl.pallas_call(
        _body_c,
        grid=(GRID,),
        in_specs=[_cnt_spec(), _row_spec(F), _full_spec((F, 128)),
                  _full_spec((1, 128)), _full_spec((128, F))],
        out_specs=_row_spec(F),
        out_shape=jax.ShapeDtypeStruct((NR, F), jnp.float32),
    )(cnt, u, w2, b2, w3)


def _tc_e(cnt, u, wmu, bmu, wls, bls):
    return pl.pallas_call(
        _body_e,
        grid=(GRID,),
        in_specs=[_cnt_spec(), _row_spec(F), _full_spec((F, 32)),
                  _full_spec((1, 32)), _full_spec((F, 32)), _full_spec((1, 32))],
        out_specs=[_row_spec(32), _row_spec(32)],
        out_shape=[jax.ShapeDtypeStruct((N, 32), jnp.float32),
                   jax.ShapeDtypeStruct((N, 32), jnp.float32)],
    )(cnt, u, wmu, bmu, wls, bls)


def kernel(x, edge_index, W1, b1, W2, b2, W3, b3, Wmu, bmu, Wls, bls):
    src = edge_index[0].astype(jnp.int32)
    dst = edge_index[1].astype(jnp.int32)
    pad = E_PAD - E
    srcs = jnp.concatenate([src, jnp.zeros((pad,), jnp.int32)]).reshape(16, NCH, CH)
    dsts = jnp.concatenate([dst, jnp.full((pad,), N, jnp.int32)]).reshape(16, NCH, CH)

    zeros16 = jnp.zeros((NR, 16), jnp.float32)
    ones_rows = jnp.ones((CH, 16), jnp.float32)
    cnt = _deg_counts(dsts, zeros16, ones_rows)

    z1 = _tc_a(cnt, x, W1)
    u1 = _agg_pass(z1, srcs, dsts)
    z2 = _tc_b(cnt, u1, b1.reshape(1, F))
    u2 = _agg_pass(z2, srcs, dsts)
    z3 = _tc_c(cnt, u2, W2, b2.reshape(1, 128), W3)
    u3 = _agg_pass(z3, srcs, dsts)
    z4 = _tc_b(cnt, u3, b3.reshape(1, F))
    u4 = _agg_pass(z4, srcs, dsts)
    mu, ls = _tc_e(cnt, u4, Wmu, bmu.reshape(1, 32), Wls, bls.reshape(1, 32))
    return (mu, ls)
